# baseline (speedup 1.0000x reference)
import functools

import jax
import jax.numpy as jnp
from jax.experimental import pallas as pl

_ANCH = {76: ((28.0, 28.0), (46.0, 45.0), (64.0, 66.0)),
         38: ((102.0, 74.0), (78.0, 115.0), (132.0, 113.0)),
         19: ((149.0, 163.0), (174.0, 268.0), (257.0, 176.0))}

_NBOX = 50


def _slog(x):
    return jnp.maximum(jnp.log(x), -100.0)


def _scale_kernel(raw_ref, rawc_ref, lab_ref, pt_ref, nt_ref, out_ref, *, size):
    ss = size * size
    stride = 608 // size
    imax = 79 // stride + 1
    ssc = size * imax
    anchors = _ANCH[size]
    lab = lab_ref[...]
    pt = pt_ref[0, 0]
    nt = nt_ref[0, 0]

    cls = lab[:, 0:1]
    bx = lab[:, 1:2]
    bw = lab[:, 3:4]
    bh = lab[:, 4:5]

    rowsum = jnp.sum(lab, axis=1, keepdims=True)
    n = jnp.sum(jnp.where(rowsum > 0.0, 1.0, 0.0))
    tvec = jax.lax.broadcasted_iota(jnp.int32, (_NBOX, 1), 0).astype(jnp.float32)
    validf = jnp.where(tvec < n, 1.0, 0.0)
    has_box = jnp.where(n > 0.0, 1.0, 0.0)

    labT = jnp.transpose(lab)
    clsr = labT[0:1, :]
    bxr = labT[1:2, :]
    byr = labT[2:3, :]
    bwr = labT[3:4, :]
    bhr = labT[4:5, :]
    trow = jax.lax.broadcasted_iota(jnp.int32, (1, _NBOX), 1).astype(jnp.float32)
    validr = jnp.where(trow < n, 1.0, 0.0)

    jr = jnp.floor(bxr / stride)
    ir = jnp.floor(clsr / stride)
    cellr = jr * size + ir
    cellc = jnp.floor(bx / stride) * size + jnp.floor(cls / stride)

    sio_sub = jax.lax.broadcasted_iota(jnp.int32, (ssc, _NBOX), 0)
    celli = (jr * imax + ir).astype(jnp.int32)
    hc = jnp.where(sio_sub == celli, 1.0, 0.0)

    cellmatch = jnp.where(cellc == cellr, 1.0, 0.0)

    sio = jax.lax.broadcasted_iota(jnp.int32, (1, ss), 1).astype(jnp.float32)
    gy = jnp.floor(sio / size)
    gx = sio - gy * size

    area_b = bw * bh
    ntab = nt * (area_b + 1e-7)
    bx1 = bx - bw * 0.5
    bx2 = bx + bw * 0.5
    by1 = lab[:, 2:3] - bh * 0.5
    by2 = lab[:, 2:3] + bh * 0.5

    ciota = jax.lax.broadcasted_iota(jnp.int32, (80, 1), 0).astype(jnp.float32)
    onehot = jnp.where(ciota == clsr, 1.0, 0.0)

    s_pos = jnp.float32(0.0)
    s_neg = jnp.float32(0.0)
    s_box = jnp.float32(0.0)
    s_bp = jnp.float32(0.0)
    s_bn = jnp.float32(0.0)
    s_cls = jnp.float32(0.0)

    for a in range(3):
        aw, ah = anchors[a]
        base = a * 85

        piour = (jnp.minimum(bwr, aw) * jnp.minimum(bhr, ah))
        piour = piour / (bwr * bhr + aw * ah - piour + 1e-7)
        maskr = jnp.where(piour > pt, 1.0, 0.0) * validr
        maskc = jnp.transpose(maskr)

        hmat = cellmatch * maskc
        tpg = jnp.max(hmat * (tvec + 1.0), axis=0, keepdims=True)
        canon = maskr * jnp.where(tpg == trow + 1.0, 1.0, 0.0)

        kraw = jax.lax.dot_general(
            onehot, hmat, (((1,), (0,)), ((), ())),
            preferred_element_type=jnp.float32)
        kset = jnp.minimum(kraw, 1.0)

        zg = jax.lax.dot_general(
            rawc_ref[base:base + 85, :], hc, (((1,), (0,)), ((), ())),
            preferred_element_type=jnp.float32)
        pxg = (jax.nn.sigmoid(zg[0:1, :]) * 1.05 - 0.025 + ir) * stride
        pyg = (jax.nn.sigmoid(zg[1:2, :]) * 1.05 - 0.025 + jr) * stride
        pwg = jnp.exp(zg[2:3, :]) * aw
        phg = jnp.exp(zg[3:4, :]) * ah
        confg = jax.nn.sigmoid(zg[4:5, :])
        pg = jax.nn.sigmoid(zg[5:85, :])

        px = (jax.nn.sigmoid(raw_ref[base + 0:base + 1, :]) * 1.05 - 0.025 + gx) * stride
        py = (jax.nn.sigmoid(raw_ref[base + 1:base + 2, :]) * 1.05 - 0.025 + gy) * stride
        pw = jnp.exp(raw_ref[base + 2:base + 3, :]) * aw
        ph = jnp.exp(raw_ref[base + 3:base + 4, :]) * ah
        conf = jax.nn.sigmoid(raw_ref[base + 4:base + 5, :])

        px1 = px - pw * 0.5
        px2 = px + pw * 0.5
        py1 = py - ph * 0.5
        py2 = py + ph * 0.5
        iw = jnp.maximum(jnp.minimum(px2, bx2) - jnp.maximum(px1, bx1), 0.0)
        ih = jnp.maximum(jnp.minimum(py2, by2) - jnp.maximum(py1, by1), 0.0)
        ai = iw * ih
        area_p = pw * ph
        exceed = jnp.where(ai * (1.0 + nt) >= nt * area_p + ntab,
                           validf, 0.0)
        cnt = jnp.sum(exceed, axis=0, keepdims=True)
        negf = jnp.where(cnt == 0.0, 1.0, 0.0) * has_box

        pxg1 = pxg - pwg * 0.5
        pxg2 = pxg + pwg * 0.5
        pyg1 = pyg - phg * 0.5
        pyg2 = pyg + phg * 0.5
        iwg = jnp.maximum(jnp.minimum(pxg2, bx2) - jnp.maximum(pxg1, bx1), 0.0)
        ihg = jnp.maximum(jnp.minimum(pyg2, by2) - jnp.maximum(pyg1, by1), 0.0)
        aig = iwg * ihg
        areapg = pwg * phg
        exceedg = jnp.where(aig * (1.0 + nt) >= nt * areapg + ntab,
                            validf, 0.0)
        cntg = jnp.sum(exceedg, axis=0, keepdims=True)
        negg = jnp.where(cntg == 0.0, 1.0, 0.0) * has_box

        tx1 = bxr - bwr * 0.5
        tx2 = bxr + bwr * 0.5
        ty1 = byr - bhr * 0.5
        ty2 = byr + bhr * 0.5
        iw2 = jnp.maximum(jnp.minimum(pxg2, tx2) - jnp.maximum(pxg1, tx1), 0.0)
        ih2 = jnp.maximum(jnp.minimum(pyg2, ty2) - jnp.maximum(pyg1, ty1), 0.0)
        ai2 = iw2 * ih2
        iou2 = ai2 / (areapg + bwr * bhr - ai2 + 1e-7)
        ow = jnp.maximum(jnp.maximum(pxg2, tx2) - jnp.minimum(pxg1, tx1), 0.0)
        oh = jnp.maximum(jnp.maximum(pyg2, ty2) - jnp.minimum(pyg1, ty1), 0.0)
        r2 = (pxg - bxr) * (pxg - bxr) + (pyg - byr) * (pyg - byr)
        c2 = ow * ow + oh * oh
        dioug = iou2 - r2 / (c2 + 1e-7)

        slc1 = _slog(1.0 - conf)
        s_bn_dense = jnp.sum(negf * (-slc1))
        slgc = _slog(confg)
        slgc1 = _slog(1.0 - confg)
        s_bp = s_bp + jnp.sum(canon * (-slgc))
        s_bn = s_bn + s_bn_dense + jnp.sum(canon * negg * (slgc1 - slgc))

        bcecg = -(kset * _slog(pg) + (1.0 - kset) * _slog(1.0 - pg))
        colsum = jnp.sum(bcecg, axis=0, keepdims=True)
        s_cls = s_cls + jnp.sum(canon * colsum)

        s_pos = s_pos + jnp.sum(canon)
        s_neg = s_neg + jnp.sum(negf)
        s_box = s_box + jnp.sum(canon * (1.0 - dioug))

    oidx = jax.lax.broadcasted_iota(jnp.int32, (1, 8), 1).astype(jnp.float32)
    vals = jnp.where(oidx == 0.0, s_pos,
           jnp.where(oidx == 1.0, s_neg,
           jnp.where(oidx == 2.0, s_box,
           jnp.where(oidx == 3.0, s_bp,
           jnp.where(oidx == 4.0, s_bn,
           jnp.where(oidx == 5.0, s_cls, 0.0))))))
    out_ref[...] = vals


def _run_scale(raw, label, pt, nt, size):
    batch = raw.shape[0]
    ss = size * size
    stride = 608 // size
    imax = 79 // stride + 1
    raw2 = raw.reshape(batch, 255, ss)
    rawc = raw[:, :, :, :imax].reshape(batch, 255, size * imax)
    out = pl.pallas_call(
        functools.partial(_scale_kernel, size=size),
        grid=(batch,),
        in_specs=[
            pl.BlockSpec((None, 255, ss), lambda b: (b, 0, 0)),
            pl.BlockSpec((None, 255, size * imax), lambda b: (b, 0, 0)),
            pl.BlockSpec((None, _NBOX, 5), lambda b: (b, 0, 0)),
            pl.BlockSpec((1, 1), lambda b: (0, 0)),
            pl.BlockSpec((1, 1), lambda b: (0, 0)),
        ],
        out_specs=pl.BlockSpec((None, 1, 8), lambda b: (b, 0, 0)),
        out_shape=jax.ShapeDtypeStruct((batch, 1, 8), jnp.float32),
    )(raw2, rawc, label, pt.reshape(1, 1), nt.reshape(1, 1))
    return jnp.sum(out.reshape(batch, 8), axis=0)


def kernel(predict_76, predict_38, predict_19, label, pos_thresh=0.2,
           neg_thresh=0.7):
    pt = jnp.asarray(pos_thresh, jnp.float32)
    nt = jnp.asarray(neg_thresh, jnp.float32)
    loss_box = jnp.zeros((), jnp.float32)
    loss_obj = jnp.zeros((), jnp.float32)
    loss_cls = jnp.zeros((), jnp.float32)
    for raw in (predict_76, predict_38, predict_19):
        size = raw.shape[2]
        batch = raw.shape[0]
        sums = _run_scale(raw, label, pt, nt, size)
        pos_cnt = sums[0]
        neg_cnt = sums[1]
        use = (pos_cnt > 0.0) & (neg_cnt > 0.0)
        lb = sums[2] / pos_cnt / batch
        lo = sums[3] / pos_cnt / batch + sums[4] / neg_cnt / batch
        lc = sums[5] / (pos_cnt * 80.0) / batch
        loss_box = loss_box + jnp.where(use, lb, 0.0)
        loss_obj = loss_obj + jnp.where(use, lo, 0.0)
        loss_cls = loss_cls + jnp.where(use, lc, 0.0)
    loss = loss_box + loss_obj + loss_cls
    return (loss, loss_box, loss_obj, loss_cls)


# speedup vs baseline: 1.0593x; 1.0593x over previous
import functools

import jax
import jax.numpy as jnp
from jax.experimental import pallas as pl

_ANCH = {76: ((28.0, 28.0), (46.0, 45.0), (64.0, 66.0)),
         38: ((102.0, 74.0), (78.0, 115.0), (132.0, 113.0)),
         19: ((149.0, 163.0), (174.0, 268.0), (257.0, 176.0))}

_NBOX = 50


def _slog(x):
    return jnp.maximum(jnp.log(x), -100.0)


def _scale_kernel(raw_ref, lab_ref, pt_ref, nt_ref, out_ref, *, size):
    ss = size * size
    stride = 608 // size
    anchors = _ANCH[size]
    lab = lab_ref[...]
    pt = pt_ref[0, 0]
    nt = nt_ref[0, 0]

    cls = lab[:, 0:1]
    bx = lab[:, 1:2]
    bw = lab[:, 3:4]
    bh = lab[:, 4:5]

    rowsum = jnp.sum(lab, axis=1, keepdims=True)
    n = jnp.sum(jnp.where(rowsum > 0.0, 1.0, 0.0))
    tvec = jax.lax.broadcasted_iota(jnp.int32, (_NBOX, 1), 0).astype(jnp.float32)
    validf = jnp.where(tvec < n, 1.0, 0.0)
    has_box = jnp.where(n > 0.0, 1.0, 0.0)

    labT = jnp.transpose(lab)
    clsr = labT[0:1, :]
    bxr = labT[1:2, :]
    byr = labT[2:3, :]
    bwr = labT[3:4, :]
    bhr = labT[4:5, :]
    trow = jax.lax.broadcasted_iota(jnp.int32, (1, _NBOX), 1).astype(jnp.float32)
    validr = jnp.where(trow < n, 1.0, 0.0)

    jr = jnp.floor(bxr / stride)
    ir = jnp.floor(clsr / stride)
    cellr = jr * size + ir
    cellc = jnp.floor(bx / stride) * size + jnp.floor(cls / stride)

    sio_sub = jax.lax.broadcasted_iota(jnp.int32, (ss, _NBOX), 0)
    celli = cellr.astype(jnp.int32)
    hc = jnp.where(sio_sub == celli, 1.0, 0.0)

    cellmatch = jnp.where(cellc == cellr, 1.0, 0.0)

    sio = jax.lax.broadcasted_iota(jnp.int32, (1, ss), 1).astype(jnp.float32)
    gy = jnp.floor(sio / size)
    gx = sio - gy * size

    area_b = bw * bh
    ntab = nt * (area_b + 1e-7)
    bx1 = bx - bw * 0.5
    bx2 = bx + bw * 0.5
    by1 = lab[:, 2:3] - bh * 0.5
    by2 = lab[:, 2:3] + bh * 0.5

    ciota = jax.lax.broadcasted_iota(jnp.int32, (80, 1), 0).astype(jnp.float32)
    onehot = jnp.where(ciota == clsr, 1.0, 0.0)

    s_pos = jnp.float32(0.0)
    s_neg = jnp.float32(0.0)
    s_box = jnp.float32(0.0)
    s_bp = jnp.float32(0.0)
    s_bn = jnp.float32(0.0)
    s_cls = jnp.float32(0.0)

    for a in range(3):
        aw, ah = anchors[a]
        base = a * 85

        piour = (jnp.minimum(bwr, aw) * jnp.minimum(bhr, ah))
        piour = piour / (bwr * bhr + aw * ah - piour + 1e-7)
        maskr = jnp.where(piour > pt, 1.0, 0.0) * validr
        maskc = jnp.transpose(maskr)

        hmat = cellmatch * maskc
        tpg = jnp.max(hmat * (tvec + 1.0), axis=0, keepdims=True)
        canon = maskr * jnp.where(tpg == trow + 1.0, 1.0, 0.0)

        kraw = jax.lax.dot_general(
            onehot, hmat, (((1,), (0,)), ((), ())),
            preferred_element_type=jnp.float32)
        kset = jnp.minimum(kraw, 1.0)

        zg = jax.lax.dot_general(
            raw_ref[base:base + 85, :], hc, (((1,), (0,)), ((), ())),
            preferred_element_type=jnp.float32)
        pxg = (jax.nn.sigmoid(zg[0:1, :]) * 1.05 - 0.025 + ir) * stride
        pyg = (jax.nn.sigmoid(zg[1:2, :]) * 1.05 - 0.025 + jr) * stride
        pwg = jnp.exp(zg[2:3, :]) * aw
        phg = jnp.exp(zg[3:4, :]) * ah
        confg = jax.nn.sigmoid(zg[4:5, :])
        pg = jax.nn.sigmoid(zg[5:85, :])

        px = (jax.nn.sigmoid(raw_ref[base + 0:base + 1, :]) * 1.05 - 0.025 + gx) * stride
        py = (jax.nn.sigmoid(raw_ref[base + 1:base + 2, :]) * 1.05 - 0.025 + gy) * stride
        pw = jnp.exp(raw_ref[base + 2:base + 3, :]) * aw
        ph = jnp.exp(raw_ref[base + 3:base + 4, :]) * ah
        conf = jax.nn.sigmoid(raw_ref[base + 4:base + 5, :])

        px1 = px - pw * 0.5
        px2 = px + pw * 0.5
        py1 = py - ph * 0.5
        py2 = py + ph * 0.5
        iw = jnp.maximum(jnp.minimum(px2, bx2) - jnp.maximum(px1, bx1), 0.0)
        ih = jnp.maximum(jnp.minimum(py2, by2) - jnp.maximum(py1, by1), 0.0)
        ai = iw * ih
        area_p = pw * ph
        exceed = jnp.where(ai * (1.0 + nt) >= nt * area_p + ntab,
                           validf, 0.0)
        cnt = jnp.sum(exceed, axis=0, keepdims=True)
        negf = jnp.where(cnt == 0.0, 1.0, 0.0) * has_box

        pxg1 = pxg - pwg * 0.5
        pxg2 = pxg + pwg * 0.5
        pyg1 = pyg - phg * 0.5
        pyg2 = pyg + phg * 0.5
        iwg = jnp.maximum(jnp.minimum(pxg2, bx2) - jnp.maximum(pxg1, bx1), 0.0)
        ihg = jnp.maximum(jnp.minimum(pyg2, by2) - jnp.maximum(pyg1, by1), 0.0)
        aig = iwg * ihg
        areapg = pwg * phg
        exceedg = jnp.where(aig * (1.0 + nt) >= nt * areapg + ntab,
                            validf, 0.0)
        cntg = jnp.sum(exceedg, axis=0, keepdims=True)
        negg = jnp.where(cntg == 0.0, 1.0, 0.0) * has_box

        tx1 = bxr - bwr * 0.5
        tx2 = bxr + bwr * 0.5
        ty1 = byr - bhr * 0.5
        ty2 = byr + bhr * 0.5
        iw2 = jnp.maximum(jnp.minimum(pxg2, tx2) - jnp.maximum(pxg1, tx1), 0.0)
        ih2 = jnp.maximum(jnp.minimum(pyg2, ty2) - jnp.maximum(pyg1, ty1), 0.0)
        ai2 = iw2 * ih2
        iou2 = ai2 / (areapg + bwr * bhr - ai2 + 1e-7)
        ow = jnp.maximum(jnp.maximum(pxg2, tx2) - jnp.minimum(pxg1, tx1), 0.0)
        oh = jnp.maximum(jnp.maximum(pyg2, ty2) - jnp.minimum(pyg1, ty1), 0.0)
        r2 = (pxg - bxr) * (pxg - bxr) + (pyg - byr) * (pyg - byr)
        c2 = ow * ow + oh * oh
        dioug = iou2 - r2 / (c2 + 1e-7)

        slc1 = _slog(1.0 - conf)
        s_bn_dense = jnp.sum(negf * (-slc1))
        slgc = _slog(confg)
        slgc1 = _slog(1.0 - confg)
        s_bp = s_bp + jnp.sum(canon * (-slgc))
        s_bn = s_bn + s_bn_dense + jnp.sum(canon * negg * (slgc1 - slgc))

        bcecg = -(kset * _slog(pg) + (1.0 - kset) * _slog(1.0 - pg))
        colsum = jnp.sum(bcecg, axis=0, keepdims=True)
        s_cls = s_cls + jnp.sum(canon * colsum)

        s_pos = s_pos + jnp.sum(canon)
        s_neg = s_neg + jnp.sum(negf)
        s_box = s_box + jnp.sum(canon * (1.0 - dioug))

    oidx = jax.lax.broadcasted_iota(jnp.int32, (1, 8), 1).astype(jnp.float32)
    vals = jnp.where(oidx == 0.0, s_pos,
           jnp.where(oidx == 1.0, s_neg,
           jnp.where(oidx == 2.0, s_box,
           jnp.where(oidx == 3.0, s_bp,
           jnp.where(oidx == 4.0, s_bn,
           jnp.where(oidx == 5.0, s_cls, 0.0))))))
    out_ref[...] = vals


def _run_scale(raw, label, pt, nt, size):
    batch = raw.shape[0]
    ss = size * size
    raw2 = raw.reshape(batch, 255, ss)
    out = pl.pallas_call(
        functools.partial(_scale_kernel, size=size),
        grid=(batch,),
        in_specs=[
            pl.BlockSpec((None, 255, ss), lambda b: (b, 0, 0)),
            pl.BlockSpec((None, _NBOX, 5), lambda b: (b, 0, 0)),
            pl.BlockSpec((1, 1), lambda b: (0, 0)),
            pl.BlockSpec((1, 1), lambda b: (0, 0)),
        ],
        out_specs=pl.BlockSpec((None, 1, 8), lambda b: (b, 0, 0)),
        out_shape=jax.ShapeDtypeStruct((batch, 1, 8), jnp.float32),
    )(raw2, label, pt.reshape(1, 1), nt.reshape(1, 1))
    return jnp.sum(out.reshape(batch, 8), axis=0)


def kernel(predict_76, predict_38, predict_19, label, pos_thresh=0.2,
           neg_thresh=0.7):
    pt = jnp.asarray(pos_thresh, jnp.float32)
    nt = jnp.asarray(neg_thresh, jnp.float32)
    loss_box = jnp.zeros((), jnp.float32)
    loss_obj = jnp.zeros((), jnp.float32)
    loss_cls = jnp.zeros((), jnp.float32)
    for raw in (predict_76, predict_38, predict_19):
        size = raw.shape[2]
        batch = raw.shape[0]
        sums = _run_scale(raw, label, pt, nt, size)
        pos_cnt = sums[0]
        neg_cnt = sums[1]
        use = (pos_cnt > 0.0) & (neg_cnt > 0.0)
        lb = sums[2] / pos_cnt / batch
        lo = sums[3] / pos_cnt / batch + sums[4] / neg_cnt / batch
        lc = sums[5] / (pos_cnt * 80.0) / batch
        loss_box = loss_box + jnp.where(use, lb, 0.0)
        loss_obj = loss_obj + jnp.where(use, lo, 0.0)
        loss_cls = loss_cls + jnp.where(use, lc, 0.0)
    loss = loss_box + loss_obj + loss_cls
    return (loss, loss_box, loss_obj, loss_cls)
